# baseline (speedup 1.0000x reference)
import functools

import jax
import jax.numpy as jnp
from jax import lax
from jax.experimental import pallas as pl
from jax.experimental.pallas import tpu as pltpu
from jax.experimental.pallas import tpu_sc as plsc

NC = 2
NS = 16
NW = NC * NS
GRP = 128


def _build_sc_gather(tot, emb, seq):
    per_w = tot // NW
    ng = per_w // GRP

    mesh = plsc.VectorSubcoreMesh(
        core_axis_name="c", subcore_axis_name="s",
        num_cores=NC, num_subcores=NS)

    @functools.partial(
        pl.kernel,
        out_type=jax.ShapeDtypeStruct((tot, emb), jnp.float32),
        mesh=mesh,
        scratch_types=[
            pltpu.VMEM((GRP,), jnp.int32),
            pltpu.VMEM((GRP,), jnp.int32),
            pltpu.VMEM((GRP, emb), jnp.float32),
            pltpu.SemaphoreType.DMA,
            pltpu.SemaphoreType.DMA,
            pltpu.SemaphoreType.DMA,
        ],
        compiler_params=pltpu.CompilerParams(use_tc_tiling_on_sc=False),
    )
    def sc_gather(table_hbm, idx_hbm, out_hbm, jv, iv, rows, s1, s2, s3):
        wid = lax.axis_index("s") * NC + lax.axis_index("c")
        base = wid * per_w

        @pl.loop(0, ng)
        def _(g):
            t0 = base + g * GRP
            for k in range(GRP // 16):
                t = lax.iota(jnp.int32, 16) + (t0 + k * 16)
                r2 = lax.shift_right_logical(t, 1)
                h = lax.bitwise_and(t, 1)
                b = lax.bitwise_and(r2, 1023)
                p = lax.shift_right_logical(r2, 10)
                j = b * seq + 2 * p + h
                jv[pl.ds(k * 16, 16)] = j
            pltpu.async_copy(idx_hbm.at[jv], iv, s1).wait()
            pltpu.async_copy(table_hbm.at[iv], rows, s2).wait()
            pltpu.async_copy(rows, out_hbm.at[pl.ds(t0, GRP)], s3).wait()

    return sc_gather


def _mm_body(x_ref, w_ref, b_ref, o_ref):
    o_ref[...] = jnp.dot(
        x_ref[...], w_ref[...], preferred_element_type=jnp.float32
    ) + b_ref[...]


def _tc_matmul(x, w, b):
    m, k = x.shape
    n = w.shape[1]
    bm = 2048
    return pl.pallas_call(
        _mm_body,
        grid=(m // bm,),
        in_specs=[
            pl.BlockSpec((bm, k), lambda i: (i, 0)),
            pl.BlockSpec((k, n), lambda i: (0, 0)),
            pl.BlockSpec((1, n), lambda i: (0, 0)),
        ],
        out_specs=pl.BlockSpec((bm, n), lambda i: (i, 0)),
        out_shape=jax.ShapeDtypeStruct((m, n), jnp.float32),
    )(x, w, b.reshape(1, n))


def kernel(inputs, table, W, b):
    batch, seq = inputs.shape
    vocab, emb = table.shape
    d_model = W.shape[1]
    tot = batch * seq

    num_pairs = seq // 2
    idx_lin = inputs.reshape(tot)
    gathered = _build_sc_gather(tot, emb, seq)(table, idx_lin)
    pairs = gathered.reshape(tot // 2, 2 * emb)
    out = _tc_matmul(pairs, W, b)
    return out.reshape(num_pairs, batch, d_model).transpose(1, 0, 2)


# speedup vs baseline: 1.3990x; 1.3990x over previous
import functools

import jax
import jax.numpy as jnp
from jax import lax
from jax.experimental import pallas as pl
from jax.experimental.pallas import tpu as pltpu
from jax.experimental.pallas import tpu_sc as plsc

NC = 2
NS = 16
NW = NC * NS
GRP = 128


def _build_sc_gather(tot, emb, seq):
    per_w = tot // NW
    ng = per_w // GRP

    mesh = plsc.VectorSubcoreMesh(
        core_axis_name="c", subcore_axis_name="s",
        num_cores=NC, num_subcores=NS)

    nslot = 4
    assert ng % 2 == 0 and (ng - 6) % nslot == 0

    @functools.partial(
        pl.kernel,
        out_type=jax.ShapeDtypeStruct((tot, emb), jnp.float32),
        mesh=mesh,
        scratch_types=[
            [pltpu.VMEM((GRP,), jnp.int32) for _ in range(nslot)],
            [pltpu.VMEM((GRP,), jnp.int32) for _ in range(nslot)],
            [pltpu.VMEM((GRP, emb), jnp.float32) for _ in range(nslot)],
            [pltpu.SemaphoreType.DMA for _ in range(nslot)],
            [pltpu.SemaphoreType.DMA for _ in range(nslot)],
            [pltpu.SemaphoreType.DMA for _ in range(nslot)],
        ],
        compiler_params=pltpu.CompilerParams(use_tc_tiling_on_sc=False),
    )
    def sc_gather(table_hbm, idx_hbm, out_hbm, jv, iv, rows, l1s, l2s, wbs):
        wid = lax.axis_index("s") * NC + lax.axis_index("c")
        base = wid * per_w

        def l1_start(g, sl):
            t0 = base + g * GRP
            for k in range(GRP // 16):
                t = lax.iota(jnp.int32, 16) + (t0 + k * 16)
                r2 = lax.shift_right_logical(t, 1)
                h = lax.bitwise_and(t, 1)
                b = lax.bitwise_and(r2, 1023)
                p = lax.shift_right_logical(r2, 10)
                jv[sl][pl.ds(k * 16, 16)] = b * seq + 2 * p + h
            pltpu.async_copy(idx_hbm.at[jv[sl]], iv[sl], l1s[sl])

        def l1_wait(sl):
            pltpu.make_async_copy(idx_hbm.at[jv[sl]], iv[sl], l1s[sl]).wait()

        def l2_start(sl):
            pltpu.async_copy(table_hbm.at[iv[sl]], rows[sl], l2s[sl])

        def l2_wait(sl):
            pltpu.make_async_copy(
                table_hbm.at[iv[sl]], rows[sl], l2s[sl]).wait()

        def wb_start(g, sl):
            pltpu.async_copy(
                rows[sl], out_hbm.at[pl.ds(base + g * GRP, GRP)], wbs[sl])

        def wb_wait(g, sl):
            pltpu.make_async_copy(
                rows[sl], out_hbm.at[pl.ds(base + g * GRP, GRP)],
                wbs[sl]).wait()

        def emit(g, sl2, sl0, sl4, static):
            ga = g + 2
            if not static or 0 <= ga < ng:
                l1_wait(sl2)
                if not static or ga - 4 >= 0:
                    wb_wait(ga - 4, sl2)
                l2_start(sl2)
            if not static or 0 <= g < ng:
                l2_wait(sl0)
                wb_start(g, sl0)
            gc = g + 4
            if not static or gc < ng:
                l1_start(gc, sl4)

        for g in range(-4, 2):
            emit(g, (g + 2) % nslot, g % nslot, (g + 4) % nslot, True)

        @pl.loop(2, ng - 4, step=nslot)
        def _(go):
            for b2 in range(nslot):
                sl = (2 + b2) % nslot
                emit(go + b2, (sl + 2) % nslot, sl, sl, False)

        for g in range(ng - 4, ng):
            emit(g, (g + 2) % nslot, g % nslot, (g + 4) % nslot, True)
        for g in range(ng - 4, ng):
            wb_wait(g, g % nslot)

    return sc_gather


def _mm_body(x_ref, w_ref, b_ref, o_ref):
    o_ref[...] = jnp.dot(
        x_ref[...], w_ref[...], preferred_element_type=jnp.float32
    ) + b_ref[...]


def _tc_matmul(x, w, b):
    m, k = x.shape
    n = w.shape[1]
    bm = 2048
    return pl.pallas_call(
        _mm_body,
        grid=(m // bm,),
        in_specs=[
            pl.BlockSpec((bm, k), lambda i: (i, 0)),
            pl.BlockSpec((k, n), lambda i: (0, 0)),
            pl.BlockSpec((1, n), lambda i: (0, 0)),
        ],
        out_specs=pl.BlockSpec((bm, n), lambda i: (i, 0)),
        out_shape=jax.ShapeDtypeStruct((m, n), jnp.float32),
    )(x, w, b.reshape(1, n))


def kernel(inputs, table, W, b):
    batch, seq = inputs.shape
    vocab, emb = table.shape
    d_model = W.shape[1]
    tot = batch * seq

    num_pairs = seq // 2
    idx_lin = inputs.reshape(tot)
    gathered = _build_sc_gather(tot, emb, seq)(table, idx_lin)
    pairs = gathered.reshape(tot // 2, 2 * emb)
    out = _tc_matmul(pairs, W, b)
    return out.reshape(num_pairs, batch, d_model).transpose(1, 0, 2)
